# baseline (speedup 1.0000x reference)
import functools

import jax
import jax.numpy as jnp
from jax import lax
from jax.experimental import pallas as pl
from jax.experimental.pallas import tpu as pltpu
from jax.experimental.pallas import tpu_sc as plsc

_N = 10000
_E = 320000
_D = 128
_T = 16

_NC = 2
_NS = 16
_NW = _NC * _NS
_EW = _E // _NW
_C = 80
_NCHUNK = _EW // _C
_NB = 4
_NI = 4


def _pre_body(x_ref, w_ref, al_ref, ar_ref, eemb_ref, ae_ref,
              h_ref, el_ref, er_ref, te_ref):
    h = jnp.dot(x_ref[...], w_ref[...], preferred_element_type=jnp.float32)
    h_ref[...] = h
    el_ref[...] = h @ al_ref[...]
    er_ref[...] = h @ ar_ref[...]
    te_ref[...] = eemb_ref[...] @ ae_ref[...]


def _pre(x, W, a_l, a_r, edge_emb, a_e):
    return pl.pallas_call(
        _pre_body,
        out_shape=[
            jax.ShapeDtypeStruct((_N, _D), jnp.float32),
            jax.ShapeDtypeStruct((_N,), jnp.float32),
            jax.ShapeDtypeStruct((_N,), jnp.float32),
            jax.ShapeDtypeStruct((_T,), jnp.float32),
        ],
    )(x, W, a_l, a_r, edge_emb, a_e)


def _sc_body(h_hbm, el_hbm, er_hbm, te_hbm, src_hbm, dst_hbm, ef_hbm,
             znd_hbm, zn_hbm,
             acc_out, den_out,
             te_v, srcb, dstb, efb, elg, erg, exb, rows, den_stage,
             acc_sh, den_sh, sem_g, sem_sr, sem_sd, sem_i):
    c = lax.axis_index("c")
    s = lax.axis_index("s")
    wid = s * _NC + c
    ebase = wid * _EW

    def fire_idx(i, bi):
        base = ebase + i * _C
        pltpu.async_copy(src_hbm.at[pl.ds(base, _C)], srcb.at[bi], sem_i.at[bi])
        pltpu.async_copy(dst_hbm.at[pl.ds(base, _C)], dstb.at[bi], sem_i.at[bi])
        pltpu.async_copy(ef_hbm.at[pl.ds(base, _C)], efb.at[bi], sem_i.at[bi])

    def wait_idx(bi):
        pltpu.make_async_copy(src_hbm.at[pl.ds(0, _C)], srcb.at[bi],
                              sem_i.at[bi]).wait()
        pltpu.make_async_copy(dst_hbm.at[pl.ds(0, _C)], dstb.at[bi],
                              sem_i.at[bi]).wait()
        pltpu.make_async_copy(ef_hbm.at[pl.ds(0, _C)], efb.at[bi],
                              sem_i.at[bi]).wait()

    def fire_gathers(bi, b):
        pltpu.async_copy(h_hbm.at[srcb.at[bi]], rows.at[b], sem_g.at[b])
        pltpu.async_copy(el_hbm.at[srcb.at[bi]], elg.at[b], sem_g.at[b])
        pltpu.async_copy(er_hbm.at[dstb.at[bi]], erg.at[b], sem_g.at[b])

    def wait_gathers(bi, b):
        pltpu.make_async_copy(h_hbm.at[srcb.at[bi]], rows.at[b],
                              sem_g.at[b]).wait()
        pltpu.make_async_copy(el_hbm.at[srcb.at[bi]], elg.at[b],
                              sem_g.at[b]).wait()
        pltpu.make_async_copy(er_hbm.at[dstb.at[bi]], erg.at[b],
                              sem_g.at[b]).wait()

    def wait_scatters(bi, b):
        pltpu.make_async_copy(rows.at[b], acc_sh.at[dstb.at[bi]],
                              sem_sr.at[b]).wait()
        pltpu.make_async_copy(exb.at[b], den_sh.at[dstb.at[bi]],
                              sem_sd.at[b]).wait()

    @pl.when(s < 10)
    def _zero():
        pltpu.sync_copy(znd_hbm.at[pl.ds(s * 1000, 1000)],
                        acc_sh.at[pl.ds(s * 1000, 1000)])
        pltpu.sync_copy(zn_hbm.at[pl.ds(s * 1000, 1000)], den_stage)
        pltpu.sync_copy(den_stage, den_sh.at[pl.ds(s * 1000, 1000)])

    pltpu.sync_copy(te_hbm, te_v)
    plsc.subcore_barrier()

    for i in range(3):
        fire_idx(i, i)
    wait_idx(0)
    fire_gathers(0, 0)
    wait_idx(1)
    fire_gathers(1, 1)

    def chunk_step(i, b, bi):
        wait_gathers(bi, b)
        for grp in range(_C // 16):
            sl = pl.ds(grp * 16, 16)
            gt = plsc.load_gather(te_v, [efb[bi, sl]])
            logit = elg[b, sl] + erg[b, sl] + gt
            logit = jnp.maximum(logit, 0.2 * logit)
            exb[b, sl] = jnp.exp(logit)

        pltpu.async_copy(exb.at[b], den_sh.at[dstb.at[bi]],
                         sem_sd.at[b], add=True)

        def scale(grp, carry2):
            ex16 = exb[b, pl.ds(grp * 16, 16)]
            for l in range(16):
                w = ex16[l]
                e = grp * 16 + l
                for r in range(_D // 16):
                    cs = pl.ds(r * 16, 16)
                    rows[b, e, cs] = rows[b, e, cs] * w
            return carry2

        lax.fori_loop(0, _C // 16, scale, 0)

        pltpu.async_copy(rows.at[b], acc_sh.at[dstb.at[bi]],
                         sem_sr.at[b], add=True)

    def step(g, carry):
        for k in range(_NI):
            i = g * _NI + k
            b = k
            bp = (k + 3) % _NB
            b2 = (k + 2) % _NB

            @pl.when(i >= 1)
            def _():
                wait_scatters(bp, bp)

            @pl.when(i + 3 <= _NCHUNK - 1)
            def _():
                fire_idx(i + 3, bp)

            @pl.when(i + 2 <= _NCHUNK - 1)
            def _():
                wait_idx(b2)
                fire_gathers(b2, b2)

            chunk_step(i, b, b)
        return carry

    lax.fori_loop(0, (_NCHUNK - 1) // _NI, step, 0)

    wait_scatters(3, 3)
    chunk_step(_NCHUNK - 1, 0, 0)
    wait_scatters(0, 0)

    plsc.subcore_barrier()

    @pl.when(s < 10)
    def _writeout():
        pltpu.sync_copy(acc_sh.at[pl.ds(s * 1000, 1000)],
                        acc_out.at[c, pl.ds(s * 1000, 1000)])
        pltpu.sync_copy(den_sh.at[pl.ds(s * 1000, 1000)], den_stage)
        pltpu.sync_copy(den_stage,
                        den_out.at[pl.ds(c * _N + s * 1000, 1000)])


@functools.cache
def _sc_call():
    return pl.kernel(
        _sc_body,
        out_type=[
            jax.ShapeDtypeStruct((_NC, _N, _D), jnp.float32),
            jax.ShapeDtypeStruct((_NC * _N,), jnp.float32),
        ],
        mesh=plsc.VectorSubcoreMesh(core_axis_name="c", subcore_axis_name="s",
                                    num_cores=_NC, num_subcores=_NS),
        compiler_params=pltpu.CompilerParams(needs_layout_passes=False),
        scratch_types=[
            pltpu.VMEM((_T,), jnp.float32),
            pltpu.VMEM((_NI, _C), jnp.int32),
            pltpu.VMEM((_NI, _C), jnp.int32),
            pltpu.VMEM((_NI, _C), jnp.int32),
            pltpu.VMEM((_NB, _C), jnp.float32),
            pltpu.VMEM((_NB, _C), jnp.float32),
            pltpu.VMEM((_NB, _C), jnp.float32),
            pltpu.VMEM((_NB, _C, _D), jnp.float32),
            pltpu.VMEM((1000,), jnp.float32),
            pltpu.VMEM_SHARED((_N, _D), jnp.float32),
            pltpu.VMEM_SHARED((_N,), jnp.float32),
            pltpu.SemaphoreType.DMA((_NB,)),
            pltpu.SemaphoreType.DMA((_NB,)),
            pltpu.SemaphoreType.DMA((_NB,)),
            pltpu.SemaphoreType.DMA((_NI,)),
        ],
    )


def _post_body(acc_ref, den_ref, b_ref, o_ref):
    den = den_ref[0, :] + den_ref[1, :] + jnp.float32(1e-9)
    acc = acc_ref[0] + acc_ref[1]
    o = acc / den[:, None] + b_ref[...][None, :]
    o_ref[...] = jnp.where(o > 0, o, jnp.exp(jnp.minimum(o, 0.0)) - 1.0)


def _post(acc, den, bias):
    return pl.pallas_call(
        _post_body,
        out_shape=jax.ShapeDtypeStruct((_N, _D), jnp.float32),
    )(acc, den, bias)


def kernel(x, edge_index, e_feat, W, edge_emb, a_l, a_r, a_e, bias):
    src = edge_index[0]
    dst = edge_index[1]
    h, el, er, te = _pre(x, W, a_l, a_r, edge_emb, a_e)
    znd = jnp.zeros((_N, _D), jnp.float32)
    zn = jnp.zeros((_N,), jnp.float32)
    acc, den = _sc_call()(h, el, er, te, src, dst, e_feat, znd, zn)
    return _post(acc, den.reshape(_NC, _N), bias)
